# Initial kernel scaffold; baseline (speedup 1.0000x reference)
import jax
import jax.numpy as jnp
from jax.experimental import pallas as pl


def kernel(w, mask):
    raise NotImplementedError("write your pallas kernel here")


# speedup vs baseline: 1.3844x; 1.3844x over previous
import functools

import jax
import jax.numpy as jnp
from jax import lax
from jax.experimental import pallas as pl
from jax.experimental.pallas import tpu as pltpu
from jax.experimental.pallas import tpu_sc as plsc

N = 4096
D = 2048
L = 16
NC = 2
NS = 16
NW = NC * NS
ROWS_PER_W = N // NW
CHUNK = L
NCHUNK = ROWS_PER_W // CHUNK

_mesh = plsc.VectorSubcoreMesh(core_axis_name="c", subcore_axis_name="s")


@functools.partial(
    pl.kernel,
    out_type=jax.ShapeDtypeStruct((N, D), jnp.float32),
    mesh=_mesh,
    compiler_params=pltpu.CompilerParams(needs_layout_passes=False),
    scratch_types=[
        pltpu.VMEM((N,), jnp.int32),
        pltpu.VMEM((CHUNK, D), jnp.float32),
        pltpu.VMEM((CHUNK, D), jnp.float32),
        pltpu.SemaphoreType.DMA,
        pltpu.SemaphoreType.DMA,
        pltpu.SemaphoreType.DMA,
        pltpu.SemaphoreType.DMA,
        pltpu.SemaphoreType.DMA,
    ],
)
def _recover(w_hbm, mask_hbm, out_hbm, mask_v, buf0, buf1,
             sem_m, sem_g0, sem_g1, sem_w0, sem_w1):
    cid = lax.axis_index("c")
    sid = lax.axis_index("s")
    wid = sid * NC + cid
    base = wid * ROWS_PER_W

    pltpu.async_copy(mask_hbm, mask_v, sem_m).wait()

    def _psum(j, acc):
        return acc + mask_v[pl.ds(j * L, L)]

    acc = lax.fori_loop(0, wid * (ROWS_PER_W // L), _psum,
                        jnp.zeros((L,), jnp.int32))
    prefix = jnp.sum(acc)

    iota = lax.iota(jnp.int32, L)

    srcs, dests, n1s = [], [], []
    run = prefix
    for c in range(NCHUNK):
        mvec = mask_v[pl.ds(base + c * CHUNK, CHUNK)]
        n1 = jnp.sum(mvec)
        keys = (1 - mvec) * CHUNK + iota
        _, spos = plsc.sort_key_val(keys, base + c * CHUNK + iota)
        dests.append(spos)
        n1s.append(n1)
        srcs.append(jnp.minimum(run + iota, N - 1))
        run = run + n1

    zero = jnp.zeros((L,), jnp.float32)
    bufs = [buf0, buf1]
    gsems = [sem_g0, sem_g1]
    wsems = [sem_w0, sem_w1]
    gh = [None] * NCHUNK
    wh = [None] * NCHUNK
    gh[0] = pltpu.async_copy(w_hbm.at[plsc.Indices(srcs[0])], bufs[0],
                             gsems[0])
    gh[1] = pltpu.async_copy(w_hbm.at[plsc.Indices(srcs[1])], bufs[1],
                             gsems[1])
    for c in range(NCHUNK):
        b = bufs[c % 2]
        gh[c].wait()

        def _zrow(r, carry, b=b):
            def _zcol(g, inner):
                for u in range(16):
                    b[r, pl.ds(g * 16 * 16 + u * L, L)] = zero
                return inner
            return lax.fori_loop(0, D // (16 * 16), _zcol, carry)

        lax.fori_loop(n1s[c], CHUNK, _zrow, 0)

        wh[c] = pltpu.async_copy(
            b, out_hbm.at[plsc.Indices(dests[c])], wsems[c % 2])
        if c + 2 < NCHUNK:
            wh[c].wait()
            gh[c + 2] = pltpu.async_copy(
                w_hbm.at[plsc.Indices(srcs[c + 2])], bufs[c % 2],
                gsems[c % 2])
    wh[NCHUNK - 2].wait()
    wh[NCHUNK - 1].wait()


def kernel(w, mask):
    return _recover(w, mask)


# speedup vs baseline: 1.4338x; 1.0357x over previous
import functools

import jax
import jax.numpy as jnp
from jax import lax
from jax.experimental import pallas as pl
from jax.experimental.pallas import tpu as pltpu
from jax.experimental.pallas import tpu_sc as plsc

N = 4096
D = 2048
L = 16
NC = 2
NS = 16
NW = NC * NS
ROWS_PER_W = N // NW
CHUNK = L
NCHUNK = ROWS_PER_W // CHUNK

_mesh = plsc.VectorSubcoreMesh(core_axis_name="c", subcore_axis_name="s")


@functools.partial(
    pl.kernel,
    out_type=jax.ShapeDtypeStruct((N, D), jnp.float32),
    mesh=_mesh,
    compiler_params=pltpu.CompilerParams(needs_layout_passes=False),
    scratch_types=[
        pltpu.VMEM((N,), jnp.int32),
        pltpu.VMEM((CHUNK, D), jnp.float32),
        pltpu.VMEM((CHUNK, D), jnp.float32),
        pltpu.VMEM((CHUNK, D), jnp.float32),
        pltpu.SemaphoreType.DMA,
        pltpu.SemaphoreType.DMA,
        pltpu.SemaphoreType.DMA,
        pltpu.SemaphoreType.DMA,
        pltpu.SemaphoreType.DMA,
        pltpu.SemaphoreType.DMA,
        pltpu.SemaphoreType.DMA,
    ],
)
def _recover(w_hbm, mask_hbm, out_hbm, mask_v, buf0, buf1, buf2,
             sem_m, sem_g0, sem_g1, sem_g2, sem_w0, sem_w1, sem_w2):
    cid = lax.axis_index("c")
    sid = lax.axis_index("s")
    wid = sid * NC + cid
    base = wid * ROWS_PER_W

    pltpu.async_copy(mask_hbm, mask_v, sem_m).wait()

    def _psum(j, acc):
        return acc + mask_v[pl.ds(j * L, L)]

    acc = lax.fori_loop(0, wid * (ROWS_PER_W // L), _psum,
                        jnp.zeros((L,), jnp.int32))
    prefix = jnp.sum(acc)

    iota = lax.iota(jnp.int32, L)

    srcs, dests, n1s = [], [], []
    run = prefix
    for c in range(NCHUNK):
        mvec = mask_v[pl.ds(base + c * CHUNK, CHUNK)]
        n1 = jnp.sum(mvec)
        keys = (1 - mvec) * CHUNK + iota
        _, spos = plsc.sort_key_val(keys, base + c * CHUNK + iota)
        dests.append(spos)
        n1s.append(n1)
        srcs.append(jnp.minimum(run + iota, N - 1))
        run = run + n1

    zero = jnp.zeros((L,), jnp.float32)
    bufs = [buf0, buf1, buf2]
    gsems = [sem_g0, sem_g1, sem_g2]
    wsems = [sem_w0, sem_w1, sem_w2]
    gh = [None] * NCHUNK
    wh = [None] * NCHUNK
    gh[0] = pltpu.async_copy(w_hbm.at[plsc.Indices(srcs[0])], bufs[0],
                             gsems[0])
    gh[1] = pltpu.async_copy(w_hbm.at[plsc.Indices(srcs[1])], bufs[1],
                             gsems[1])
    for c in range(NCHUNK):
        b = bufs[c % 3]
        if c + 2 < NCHUNK:
            if c >= 1:
                wh[c - 1].wait()
            gh[c + 2] = pltpu.async_copy(
                w_hbm.at[plsc.Indices(srcs[c + 2])], bufs[(c + 2) % 3],
                gsems[(c + 2) % 3])
        gh[c].wait()

        def _zrow(r, carry, b=b):
            def _zcol(g, inner):
                for u in range(16):
                    b[r, pl.ds(g * 16 * 16 + u * L, L)] = zero
                return inner
            return lax.fori_loop(0, D // (16 * 16), _zcol, carry)

        lax.fori_loop(n1s[c], CHUNK, _zrow, 0)

        wh[c] = pltpu.async_copy(
            b, out_hbm.at[plsc.Indices(dests[c])], wsems[c % 3])
    wh[NCHUNK - 3].wait()
    wh[NCHUNK - 2].wait()
    wh[NCHUNK - 1].wait()


def kernel(w, mask):
    return _recover(w, mask)


# speedup vs baseline: 1.6210x; 1.1306x over previous
import functools

import jax
import jax.numpy as jnp
from jax import lax
from jax.experimental import pallas as pl
from jax.experimental.pallas import tpu as pltpu
from jax.experimental.pallas import tpu_sc as plsc

N = 4096
D = 2048
L = 16
NC = 2
NS = 16
NW = NC * NS
ROWS_PER_W = N // NW
NGROUP = ROWS_PER_W // L
ROW_BYTES = D * 4
BUF_BYTES = L * ROW_BYTES

_mesh = plsc.VectorSubcoreMesh(core_axis_name="c", subcore_axis_name="s")


@functools.partial(
    pl.kernel,
    out_type=jax.ShapeDtypeStruct((N, D), jnp.float32),
    mesh=_mesh,
    compiler_params=pltpu.CompilerParams(needs_layout_passes=False),
    scratch_types=[
        pltpu.VMEM((N,), jnp.int32),
        pltpu.VMEM((ROWS_PER_W + L,), jnp.int32),
        pltpu.VMEM((ROWS_PER_W + L,), jnp.int32),
        pltpu.VMEM((L, D), jnp.float32),
        pltpu.VMEM((L, D), jnp.float32),
        pltpu.VMEM((L, D), jnp.float32),
        pltpu.SemaphoreType.DMA,
        pltpu.SemaphoreType.DMA,
        pltpu.SemaphoreType.DMA,
        pltpu.SemaphoreType.DMA,
        pltpu.SemaphoreType.DMA,
        pltpu.SemaphoreType.DMA,
    ],
)
def _recover(w_hbm, mask_hbm, out_hbm, mask_v, pos1_v, pos0_v,
             buf0, buf1, zbuf, sem_m, sem_g0, sem_g1, sem_w0, sem_w1,
             sem_zs):
    cid = lax.axis_index("c")
    sid = lax.axis_index("s")
    wid = sid * NC + cid
    base = wid * ROWS_PER_W

    hm = pltpu.async_copy(mask_hbm, mask_v, sem_m)

    zvec = jnp.zeros((L,), jnp.float32)

    def _zb(r, carry):
        def _zc(g, inner):
            for u in range(16):
                zbuf[r, pl.ds(g * 256 + u * L, L)] = zvec
            return inner
        return lax.fori_loop(0, D // 256, _zc, carry)

    lax.fori_loop(0, L, _zb, 0)
    hm.wait()

    def _psum(j, acc):
        return acc + mask_v[pl.ds(j * L, L)]

    acc = lax.fori_loop(0, wid * (ROWS_PER_W // L), _psum,
                        jnp.zeros((L,), jnp.int32))
    prefix = jnp.sum(acc)

    iota = lax.iota(jnp.int32, L)

    def _build(g, carry):
        cnt1, cnt0, last1, last0 = carry
        pos = base + g * L + iota
        mvec = mask_v[pl.ds(base + g * L, L)]
        m1 = mvec != 0
        n1 = jnp.sum(mvec)
        plsc.store_compressed(pos1_v.at[pl.ds(cnt1, L)], pos, mask=m1)
        plsc.store_compressed(pos0_v.at[pl.ds(cnt0, L)], pos, mask=~m1)
        last1 = jnp.maximum(last1, jnp.max(jnp.where(m1, pos, -1)))
        last0 = jnp.maximum(last0, jnp.max(jnp.where(m1, -1, pos)))
        return cnt1 + n1, cnt0 + (L - n1), last1, last0

    cnt1, cnt0, last1, last0 = lax.fori_loop(
        0, NGROUP, _build,
        (jnp.int32(0), jnp.int32(0), jnp.int32(-1), jnp.int32(-1)))
    pos1_v[pl.ds(cnt1, L)] = jnp.broadcast_to(last1, (L,))
    pos0_v[pl.ds(cnt0, L)] = jnp.broadcast_to(last0, (L,))

    nd = (cnt1 + (L - 1)) // L
    nz = (cnt0 + (L - 1)) // L

    def _zfire(d, carry):
        dest = pos0_v[pl.ds(d * L, L)]
        pltpu.async_copy(zbuf, out_hbm.at[plsc.Indices(dest)], sem_zs)
        return carry

    lax.fori_loop(0, nz, _zfire, 0)

    bufs = [buf0, buf1]
    gsems = [sem_g0, sem_g1]
    wsems = [sem_w0, sem_w1]
    last_src = prefix + jnp.maximum(cnt1 - 1, 0)

    def _wait_gather(p):
        pltpu.make_async_copy(w_hbm.at[pl.ds(0, L)], bufs[p], gsems[p]).wait()

    def _wait_scatter(p):
        pltpu.make_async_copy(bufs[p], out_hbm.at[pl.ds(0, L)],
                              wsems[p]).wait()

    def _data(d, carry):
        for p in range(2):
            @pl.when(d % 2 == p)
            def _(p=p):
                @pl.when(d < nd)
                def _():
                    @pl.when(d >= 2)
                    def _():
                        _wait_scatter(p)
                    src = jnp.minimum(prefix + d * L + iota, last_src)
                    pltpu.async_copy(w_hbm.at[plsc.Indices(src)], bufs[p],
                                     gsems[p])

                @pl.when(d >= 1)
                def _(p=1 - p):
                    _wait_gather(p)
                    dest = pos1_v[pl.ds((d - 1) * L, L)]
                    pltpu.async_copy(bufs[p],
                                     out_hbm.at[plsc.Indices(dest)],
                                     wsems[p])
        return carry

    lax.fori_loop(0, nd + 1, _data, 0)

    @pl.when(nd >= 1)
    def _():
        _wait_scatter(0)

    @pl.when(nd >= 2)
    def _():
        _wait_scatter(1)

    def _zdrain(d, carry):
        pltpu.make_async_copy(w_hbm.at[pl.ds(0, L)], zbuf, sem_zs).wait()
        return carry

    lax.fori_loop(0, nz, _zdrain, 0)


def kernel(w, mask):
    return _recover(w, mask)


# speedup vs baseline: 1.6648x; 1.0270x over previous
import functools

import jax
import jax.numpy as jnp
from jax import lax
from jax.experimental import pallas as pl
from jax.experimental.pallas import tpu as pltpu
from jax.experimental.pallas import tpu_sc as plsc

N = 4096
D = 2048
L = 16
NC = 2
NS = 16
NW = NC * NS
ROWS_PER_W = N // NW
NGROUP = ROWS_PER_W // L
ROW_BYTES = D * 4
BUF_BYTES = L * ROW_BYTES

_mesh = plsc.VectorSubcoreMesh(core_axis_name="c", subcore_axis_name="s")


@functools.partial(
    pl.kernel,
    out_type=jax.ShapeDtypeStruct((N, D), jnp.float32),
    mesh=_mesh,
    compiler_params=pltpu.CompilerParams(needs_layout_passes=False),
    scratch_types=[
        pltpu.VMEM((N,), jnp.int32),
        pltpu.VMEM((ROWS_PER_W + L,), jnp.int32),
        pltpu.VMEM((ROWS_PER_W + L,), jnp.int32),
        pltpu.VMEM((L, D), jnp.float32),
        pltpu.VMEM((L, D), jnp.float32),
        pltpu.VMEM((L, D), jnp.float32),
        pltpu.SemaphoreType.DMA,
        pltpu.SemaphoreType.DMA,
        pltpu.SemaphoreType.DMA,
        pltpu.SemaphoreType.DMA,
        pltpu.SemaphoreType.DMA,
        pltpu.SemaphoreType.DMA,
    ],
)
def _recover(w_hbm, mask_hbm, out_hbm, mask_v, pos1_v, pos0_v,
             buf0, buf1, zbuf, sem_m, sem_g0, sem_g1, sem_w0, sem_w1,
             sem_zs):
    cid = lax.axis_index("c")
    sid = lax.axis_index("s")
    wid = sid * NC + cid
    base = wid * ROWS_PER_W

    pltpu.async_copy(mask_hbm, mask_v, sem_m).wait()

    def _psum(j, acc):
        for u in range(ROWS_PER_W // L):
            acc = acc + mask_v[pl.ds((j * (ROWS_PER_W // L) + u) * L, L)]
        return acc

    acc = lax.fori_loop(0, wid, _psum, jnp.zeros((L,), jnp.int32))
    prefix = jnp.sum(acc)

    iota = lax.iota(jnp.int32, L)

    def _build(g, carry):
        cnt1, cnt0, last1, last0 = carry
        pos = base + g * L + iota
        mvec = mask_v[pl.ds(base + g * L, L)]
        m1 = mvec != 0
        n1 = jnp.sum(mvec)
        plsc.store_compressed(pos1_v.at[pl.ds(cnt1, L)], pos, mask=m1)
        plsc.store_compressed(pos0_v.at[pl.ds(cnt0, L)], pos, mask=~m1)
        last1 = jnp.maximum(last1, jnp.max(jnp.where(m1, pos, -1)))
        last0 = jnp.maximum(last0, jnp.max(jnp.where(m1, -1, pos)))
        return cnt1 + n1, cnt0 + (L - n1), last1, last0

    cnt1, cnt0, last1, last0 = lax.fori_loop(
        0, NGROUP, _build,
        (jnp.int32(0), jnp.int32(0), jnp.int32(-1), jnp.int32(-1)))
    pos1_v[pl.ds(cnt1, L)] = jnp.broadcast_to(last1, (L,))
    pos0_v[pl.ds(cnt0, L)] = jnp.broadcast_to(last0, (L,))

    nd = (cnt1 + (L - 1)) // L
    nz = (cnt0 + (L - 1)) // L

    bufs = [buf0, buf1]
    gsems = [sem_g0, sem_g1]
    wsems = [sem_w0, sem_w1]
    last_src = prefix + jnp.maximum(cnt1 - 1, 0)

    def _gather(d, p):
        src = jnp.minimum(prefix + d * L + iota, last_src)
        pltpu.async_copy(w_hbm.at[plsc.Indices(src)], bufs[p], gsems[p])

    @pl.when(nd >= 1)
    def _():
        _gather(0, 0)

    @pl.when(nd >= 2)
    def _():
        _gather(1, 1)

    zvec = jnp.zeros((L,), jnp.float32)

    def _zb(r, carry):
        def _zc(g, inner):
            for u in range(16):
                zbuf[r, pl.ds(g * 256 + u * L, L)] = zvec
            return inner
        return lax.fori_loop(0, D // 256, _zc, carry)

    lax.fori_loop(0, L, _zb, 0)

    def _zfire(d, carry):
        dest = pos0_v[pl.ds(d * L, L)]
        pltpu.async_copy(zbuf, out_hbm.at[plsc.Indices(dest)], sem_zs)
        return carry

    lax.fori_loop(0, nz, _zfire, 0)

    def _wait_gather(p):
        pltpu.make_async_copy(w_hbm.at[pl.ds(0, L)], bufs[p], gsems[p]).wait()

    def _wait_scatter(p):
        pltpu.make_async_copy(bufs[p], out_hbm.at[pl.ds(0, L)],
                              wsems[p]).wait()

    def _data(d, carry):
        for p in range(2):
            @pl.when(d % 2 == p)
            def _(p=p):
                @pl.when(jnp.logical_and(d >= 2, d < nd))
                def _():
                    _wait_scatter(p)
                    _gather(d, p)

                @pl.when(d >= 1)
                def _(p=1 - p):
                    _wait_gather(p)
                    dest = pos1_v[pl.ds((d - 1) * L, L)]
                    pltpu.async_copy(bufs[p],
                                     out_hbm.at[plsc.Indices(dest)],
                                     wsems[p])
        return carry

    lax.fori_loop(0, nd + 1, _data, 0)

    @pl.when(nd >= 1)
    def _():
        _wait_scatter(0)

    @pl.when(nd >= 2)
    def _():
        _wait_scatter(1)

    def _zdrain(d, carry):
        pltpu.make_async_copy(w_hbm.at[pl.ds(0, L)], zbuf, sem_zs).wait()
        return carry

    lax.fori_loop(0, nz, _zdrain, 0)


def kernel(w, mask):
    return _recover(w, mask)


# speedup vs baseline: 1.6742x; 1.0056x over previous
import functools

import jax
import jax.numpy as jnp
from jax import lax
from jax.experimental import pallas as pl
from jax.experimental.pallas import tpu as pltpu
from jax.experimental.pallas import tpu_sc as plsc

N = 4096
D = 2048
L = 16
NC = 2
NS = 16
NW = NC * NS
ROWS_PER_W = N // NW
NGROUP = ROWS_PER_W // L
ROW_BYTES = D * 4
BUF_BYTES = L * ROW_BYTES

_mesh = plsc.VectorSubcoreMesh(core_axis_name="c", subcore_axis_name="s")


@functools.partial(
    pl.kernel,
    out_type=jax.ShapeDtypeStruct((N, D), jnp.float32),
    mesh=_mesh,
    compiler_params=pltpu.CompilerParams(needs_layout_passes=False),
    scratch_types=[
        pltpu.VMEM((N,), jnp.int32),
        pltpu.VMEM((ROWS_PER_W + L,), jnp.int32),
        pltpu.VMEM((ROWS_PER_W + L,), jnp.int32),
        pltpu.VMEM((L, D), jnp.float32),
        pltpu.VMEM((L, D), jnp.float32),
        pltpu.VMEM((L, D), jnp.float32),
        pltpu.SemaphoreType.DMA,
        pltpu.SemaphoreType.DMA,
        pltpu.SemaphoreType.DMA,
        pltpu.SemaphoreType.DMA,
        pltpu.SemaphoreType.DMA,
        pltpu.SemaphoreType.DMA,
    ],
)
def _recover(w_hbm, mask_hbm, out_hbm, mask_v, pos1_v, pos0_v,
             buf0, buf1, zbuf, sem_m, sem_g0, sem_g1, sem_w0, sem_w1,
             sem_zs):
    cid = lax.axis_index("c")
    sid = lax.axis_index("s")
    wid = sid * NC + cid
    base = wid * ROWS_PER_W

    pltpu.async_copy(mask_hbm, mask_v, sem_m).wait()

    def _psum(j, acc):
        for u in range(ROWS_PER_W // L):
            acc = acc + mask_v[pl.ds((j * (ROWS_PER_W // L) + u) * L, L)]
        return acc

    acc = lax.fori_loop(0, wid, _psum, jnp.zeros((L,), jnp.int32))
    prefix = jnp.sum(acc)

    iota = lax.iota(jnp.int32, L)

    def _build(g, carry):
        cnt1, cnt0, last1, last0 = carry
        pos = base + g * L + iota
        mvec = mask_v[pl.ds(base + g * L, L)]
        m1 = mvec != 0
        n1 = jnp.sum(mvec)
        plsc.store_compressed(pos1_v.at[pl.ds(cnt1, L)], pos, mask=m1)
        plsc.store_compressed(pos0_v.at[pl.ds(cnt0, L)], pos, mask=~m1)
        last1 = jnp.maximum(last1, jnp.max(jnp.where(m1, pos, -1)))
        last0 = jnp.maximum(last0, jnp.max(jnp.where(m1, -1, pos)))
        return cnt1 + n1, cnt0 + (L - n1), last1, last0

    cnt1, cnt0, last1, last0 = lax.fori_loop(
        0, NGROUP, _build,
        (jnp.int32(0), jnp.int32(0), jnp.int32(-1), jnp.int32(-1)))
    pos1_v[pl.ds(cnt1, L)] = jnp.broadcast_to(last1, (L,))
    pos0_v[pl.ds(cnt0, L)] = jnp.broadcast_to(last0, (L,))

    nd = (cnt1 + (L - 1)) // L
    nz = (cnt0 + (L - 1)) // L

    bufs = [buf0, buf1]
    gsems = [sem_g0, sem_g1]
    wsems = [sem_w0, sem_w1]
    last_src = prefix + jnp.maximum(cnt1 - 1, 0)

    def _gather(d, p):
        src = jnp.minimum(prefix + d * L + iota, last_src)
        pltpu.async_copy(w_hbm.at[plsc.Indices(src)], bufs[p], gsems[p])

    @pl.when(nd >= 1)
    def _():
        _gather(0, 0)

    @pl.when(nd >= 2)
    def _():
        _gather(1, 1)

    zvec = jnp.zeros((L,), jnp.float32)

    def _zb(r, carry):
        def _zc(g, inner):
            for u in range(16):
                zbuf[r, pl.ds(g * 256 + u * L, L)] = zvec
            return inner
        return lax.fori_loop(0, D // 256, _zc, carry)

    lax.fori_loop(0, L, _zb, 0)

    def _zfire(d, carry):
        dest = pos0_v[pl.ds(d * L, L)]
        pltpu.async_copy(zbuf, out_hbm.at[plsc.Indices(dest)], sem_zs,
                         priority=1)
        return carry

    lax.fori_loop(0, nz, _zfire, 0)

    def _wait_gather(p):
        pltpu.make_async_copy(w_hbm.at[pl.ds(0, L)], bufs[p], gsems[p]).wait()

    def _wait_scatter(p):
        pltpu.make_async_copy(bufs[p], out_hbm.at[pl.ds(0, L)],
                              wsems[p]).wait()

    def _data(d, carry):
        for p in range(2):
            @pl.when(d % 2 == p)
            def _(p=p):
                @pl.when(jnp.logical_and(d >= 2, d < nd))
                def _():
                    _wait_scatter(p)
                    _gather(d, p)

                @pl.when(d >= 1)
                def _(p=1 - p):
                    _wait_gather(p)
                    dest = pos1_v[pl.ds((d - 1) * L, L)]
                    pltpu.async_copy(bufs[p],
                                     out_hbm.at[plsc.Indices(dest)],
                                     wsems[p])
        return carry

    lax.fori_loop(0, nd + 1, _data, 0)

    @pl.when(nd >= 1)
    def _():
        _wait_scatter(0)

    @pl.when(nd >= 2)
    def _():
        _wait_scatter(1)

    def _zdrain(d, carry):
        pltpu.make_async_copy(w_hbm.at[pl.ds(0, L)], zbuf, sem_zs).wait()
        return carry

    lax.fori_loop(0, nz, _zdrain, 0)


def kernel(w, mask):
    return _recover(w, mask)
